# baseline (speedup 1.0000x reference)
import functools

import jax
import jax.numpy as jnp
from jax import lax
from jax.experimental import pallas as pl
from jax.experimental.pallas import tpu as pltpu
from jax.experimental.pallas import tpu_sc as plsc

_IGNORE = 255
_THRESH = 0.7
_MIN_KEPT = 100000
_K = _MIN_KEPT + 1
_C = 19
_B, _H, _W = 8, 512, 512
_CH = 128
_STEPS = _B * (_H // _CH)
_N = _B * _H * _W

_NW = 32
_LANES = 16
_WSL = _N // _NW
_CHUNK = 16384
_E1 = 4096 * _LANES
_E3 = 256 * _LANES


def _nll_kernel(pred_ref, tgt_ref, tfloor_ref, keys_ref, stats_ref):
    b = pl.program_id(0)
    h = pl.program_id(1)

    tgt = tgt_ref[0]
    x0 = pred_ref[0, 0]
    m = x0
    for c in range(1, _C):
        m = jnp.maximum(m, pred_ref[0, c])
    z = jnp.exp(x0 - m)
    tl = jnp.where(tgt == 0, x0, 0.0)
    for c in range(1, _C):
        xc = pred_ref[0, c]
        z = z + jnp.exp(xc - m)
        tl = tl + jnp.where(tgt == c, xc, 0.0)
    nll = m + jnp.log(z) - tl
    valid = tgt != _IGNORE
    keys_ref[0] = jnp.where(valid, nll, -1.0)

    tfloor = tfloor_ref[0]
    nv = jnp.sum(valid.astype(jnp.float32))
    asum = jnp.sum(jnp.where(valid, nll, 0.0))
    tfmask = jnp.logical_and(valid, nll >= tfloor)
    tfcnt = jnp.sum(tfmask.astype(jnp.float32))
    tfsum = jnp.sum(jnp.where(tfmask, nll, 0.0))

    @pl.when(jnp.logical_and(b == 0, h == 0))
    def _init():
        stats_ref[0] = 0.0
        stats_ref[1] = 0.0
        stats_ref[2] = 0.0
        stats_ref[3] = 0.0

    stats_ref[0] += nv
    stats_ref[1] += asum
    stats_ref[2] += tfcnt
    stats_ref[3] += tfsum


def _run_nll(pred, target, tfloor):
    return pl.pallas_call(
        _nll_kernel,
        grid=(_B, _H // _CH),
        in_specs=[
            pl.BlockSpec((1, _C, _CH, _W), lambda b, h: (b, 0, h, 0)),
            pl.BlockSpec((1, _CH, _W), lambda b, h: (b, h, 0)),
            pl.BlockSpec(memory_space=pltpu.SMEM),
        ],
        out_specs=[
            pl.BlockSpec((1, _CH, _W), lambda b, h: (b * (_H // _CH) + h, 0, 0)),
            pl.BlockSpec(memory_space=pltpu.SMEM),
        ],
        out_shape=[
            jax.ShapeDtypeStruct((_STEPS, _CH, _W), jnp.float32),
            jax.ShapeDtypeStruct((4,), jnp.float32),
        ],
    )(pred, target, tfloor)


_sc_mesh = plsc.VectorSubcoreMesh(core_axis_name="c", subcore_axis_name="s")
_sc_params = pltpu.CompilerParams(needs_layout_passes=False)


def _worker_id():
    return lax.axis_index("s") * 2 + lax.axis_index("c")


def _scan_chunks(keys_hbm, keys_v, wid, body):
    lane = lax.iota(jnp.int32, _LANES)
    for ch in range(_WSL // _CHUNK):
        base = wid * _WSL + ch * _CHUNK
        pltpu.sync_copy(keys_hbm.at[pl.ds(base, _CHUNK)], keys_v)

        def inner(i, carry):
            v = keys_v[pl.ds(i * _LANES, _LANES)]
            ib = lax.bitcast_convert_type(v, jnp.int32)
            return body(v, ib, lane, carry)

        _ = lax.fori_loop(0, _CHUNK // _LANES, inner, jnp.int32(0))


@functools.partial(
    pl.kernel, mesh=_sc_mesh, compiler_params=_sc_params,
    out_type=jax.ShapeDtypeStruct((_NW, _E1), jnp.int32),
    scratch_types=[
        pltpu.VMEM((_E1,), jnp.int32),
        pltpu.VMEM((_CHUNK,), jnp.float32),
    ],
)
def _sc_scan1(keys_hbm, zeros_hbm, hist_out, hist_v, keys_v):
    wid = _worker_id()
    pltpu.sync_copy(zeros_hbm, hist_v)
    ones = jnp.ones((_LANES,), jnp.int32)

    def body(v, ib, lane, carry):
        valid = ib >= 0
        bin1 = lax.shift_right_logical(ib, 20)
        idx = lax.shift_left(bin1, 4) + lane
        idx = jnp.where(valid, idx, 0)
        plsc.addupdate_scatter(hist_v, [idx], ones, mask=valid)
        return carry

    _scan_chunks(keys_hbm, keys_v, wid, body)
    pltpu.sync_copy(hist_v, hist_out.at[wid])


@functools.partial(
    pl.kernel, mesh=_sc_mesh, compiler_params=_sc_params,
    out_type=jax.ShapeDtypeStruct((_NW, _E1), jnp.int32),
    scratch_types=[
        pltpu.VMEM((_E1,), jnp.int32),
        pltpu.VMEM((_CHUNK,), jnp.float32),
        pltpu.VMEM((_LANES,), jnp.int32),
    ],
)
def _sc_scan2(keys_hbm, zeros_hbm, params_hbm, hist_out, hist_v, keys_v, par_v):
    wid = _worker_id()
    pltpu.sync_copy(zeros_hbm, hist_v)
    pltpu.sync_copy(params_hbm, par_v)
    b1 = par_v[...][0]
    ones = jnp.ones((_LANES,), jnp.int32)

    def body(v, ib, lane, carry):
        valid = ib >= 0
        bin1 = lax.shift_right_logical(ib, 20)
        sel = jnp.logical_and(valid, bin1 == b1)
        bin2 = jnp.bitwise_and(lax.shift_right_logical(ib, 8), 0xFFF)
        idx = lax.shift_left(bin2, 4) + lane
        idx = jnp.where(sel, idx, 0)
        plsc.addupdate_scatter(hist_v, [idx], ones, mask=sel)
        return carry

    _scan_chunks(keys_hbm, keys_v, wid, body)
    pltpu.sync_copy(hist_v, hist_out.at[wid])


@functools.partial(
    pl.kernel, mesh=_sc_mesh, compiler_params=_sc_params,
    out_type=[
        jax.ShapeDtypeStruct((_NW, _E3), jnp.int32),
        jax.ShapeDtypeStruct((_NW, _LANES), jnp.float32),
        jax.ShapeDtypeStruct((_NW, _LANES), jnp.int32),
    ],
    scratch_types=[
        pltpu.VMEM((_E3,), jnp.int32),
        pltpu.VMEM((_CHUNK,), jnp.float32),
        pltpu.VMEM((_LANES,), jnp.int32),
        pltpu.VMEM((_LANES,), jnp.float32),
        pltpu.VMEM((_LANES,), jnp.int32),
    ],
)
def _sc_scan3(keys_hbm, zeros3_hbm, params_hbm, hist_out, gsum_out, gcnt_out,
              hist_v, keys_v, par_v, gsum_v, gcnt_v):
    wid = _worker_id()
    pltpu.sync_copy(zeros3_hbm, hist_v)
    pltpu.sync_copy(params_hbm, par_v)
    p24 = par_v[...][0]
    ones = jnp.ones((_LANES,), jnp.int32)
    gsum_v[...] = jnp.zeros((_LANES,), jnp.float32)
    gcnt_v[...] = jnp.zeros((_LANES,), jnp.int32)

    def body(v, ib, lane, carry):
        valid = ib >= 0
        top24 = lax.shift_right_logical(ib, 8)
        sel = jnp.logical_and(valid, top24 == p24)
        bin3 = jnp.bitwise_and(ib, 0xFF)
        idx = lax.shift_left(bin3, 4) + lane
        idx = jnp.where(sel, idx, 0)
        plsc.addupdate_scatter(hist_v, [idx], ones, mask=sel)
        above = jnp.logical_and(valid, top24 > p24)
        gsum_v[...] += jnp.where(above, v, 0.0)
        gcnt_v[...] += jnp.where(above, 1, 0)
        return carry

    _scan_chunks(keys_hbm, keys_v, wid, body)
    pltpu.sync_copy(hist_v, hist_out.at[wid])
    pltpu.sync_copy(gsum_v, gsum_out.at[wid])
    pltpu.sync_copy(gcnt_v, gcnt_out.at[wid])


def _bin_index_grid(shape, nbins):
    cols = lax.broadcasted_iota(jnp.int32, shape, 1)
    return lax.shift_right_logical(cols, 4), cols


def _count_ge(hist, binidx, cand):
    return jnp.sum(jnp.where(binidx >= cand, hist, 0))


def _locate_bits(hist, binidx, nbits, k):
    def body(i, v):
        bit = lax.shift_left(jnp.int32(1), nbits - 1 - i)
        cand = v | bit
        cnt = _count_ge(hist, binidx, cand)
        return jnp.where(cnt >= k, cand, v)

    return lax.fori_loop(0, nbits, body, jnp.int32(0))


def _locate1_kernel(hist_ref, out_ref):
    hist = hist_ref[...]
    binidx, _ = _bin_index_grid(hist.shape, 4096)
    k = jnp.int32(_K)
    b1 = _locate_bits(hist, binidx, 12, k)
    cnt_above = _count_ge(hist, binidx, b1 + 1)
    out_ref[0] = b1
    out_ref[1] = k - cnt_above


def _locate2_kernel(hist_ref, par_ref, out_ref):
    hist = hist_ref[...]
    binidx, _ = _bin_index_grid(hist.shape, 4096)
    b1 = par_ref[0]
    k2 = par_ref[1]
    b2 = _locate_bits(hist, binidx, 12, k2)
    cnt_above = _count_ge(hist, binidx, b2 + 1)
    out_ref[0] = lax.shift_left(b1, 12) | b2
    out_ref[1] = k2 - cnt_above


def _final_kernel(hist_ref, gsum_ref, gcnt_ref, par_ref, stats_ref, out_ref):
    hist = hist_ref[...]
    binidx, _ = _bin_index_grid(hist.shape, 256)
    p24 = par_ref[0]
    k3 = par_ref[1]
    b3 = _locate_bits(hist, binidx, 8, k3)

    histf = hist.astype(jnp.float32)
    inbin_mask = jnp.logical_and(binidx >= b3, hist > 0)
    inbin_cnt = jnp.sum(jnp.where(inbin_mask, histf, 0.0))
    vals = lax.bitcast_convert_type(
        lax.shift_left(p24, 8) | binidx, jnp.float32)
    inbin_sum = jnp.sum(jnp.where(inbin_mask, vals * histf, 0.0))

    gt_cnt = jnp.sum(gcnt_ref[...]).astype(jnp.float32)
    gt_sum = jnp.sum(gsum_ref[...])

    kept_cnt = jnp.maximum(gt_cnt + inbin_cnt, 1.0)
    kept_sum = gt_sum + inbin_sum
    ohem_mean = kept_sum / kept_cnt

    num_valid = stats_ref[0]
    all_sum = stats_ref[1]
    tf_cnt = stats_ref[2]
    tf_sum = stats_ref[3]
    floor_mean = tf_sum / jnp.maximum(tf_cnt, 1.0)
    all_mean = all_sum / jnp.maximum(num_valid, 1.0)

    kept_mean = jnp.where(tf_cnt >= jnp.float32(_K), ohem_mean, floor_mean)
    out_ref[0] = jnp.where(num_valid > jnp.float32(_MIN_KEPT),
                           kept_mean, all_mean)


def _run_locate1(hist):
    return pl.pallas_call(
        _locate1_kernel,
        in_specs=[pl.BlockSpec((_NW, _E1), lambda: (0, 0))],
        out_specs=pl.BlockSpec(memory_space=pltpu.SMEM),
        out_shape=jax.ShapeDtypeStruct((16,), jnp.int32),
    )(hist)


def _run_locate2(hist, params):
    return pl.pallas_call(
        _locate2_kernel,
        in_specs=[
            pl.BlockSpec((_NW, _E1), lambda: (0, 0)),
            pl.BlockSpec(memory_space=pltpu.SMEM),
        ],
        out_specs=pl.BlockSpec(memory_space=pltpu.SMEM),
        out_shape=jax.ShapeDtypeStruct((16,), jnp.int32),
    )(hist, params)


def _run_final(hist3, gsum, gcnt, params, stats):
    return pl.pallas_call(
        _final_kernel,
        in_specs=[
            pl.BlockSpec((_NW, _E3), lambda: (0, 0)),
            pl.BlockSpec((_NW, _LANES), lambda: (0, 0)),
            pl.BlockSpec((_NW, _LANES), lambda: (0, 0)),
            pl.BlockSpec(memory_space=pltpu.SMEM),
            pl.BlockSpec(memory_space=pltpu.SMEM),
        ],
        out_specs=pl.BlockSpec(memory_space=pltpu.SMEM),
        out_shape=jax.ShapeDtypeStruct((1,), jnp.float32),
    )(hist3, gsum, gcnt, params, stats)


@jax.jit
def kernel(pred, target):
    tfloor = -jnp.log(jnp.float32(_THRESH)).reshape(1)
    keys3d, stats = _run_nll(pred, target, tfloor)
    keys = keys3d.reshape(_N)

    zeros1 = jnp.zeros((_E1,), jnp.int32)
    zeros3 = jnp.zeros((_E3,), jnp.int32)

    hist1 = _sc_scan1(keys, zeros1)
    par1 = _run_locate1(hist1)
    hist2 = _sc_scan2(keys, zeros1, par1)
    par2 = _run_locate2(hist2, par1)
    hist3, gsum, gcnt = _sc_scan3(keys, zeros3, par2)
    out = _run_final(hist3, gsum, gcnt, par2, stats)
    return out[0]


# speedup vs baseline: 1.4595x; 1.4595x over previous
import functools

import jax
import jax.numpy as jnp
from jax import lax
from jax.experimental import pallas as pl
from jax.experimental.pallas import tpu as pltpu
from jax.experimental.pallas import tpu_sc as plsc

_IGNORE = 255
_THRESH = 0.7
_MIN_KEPT = 100000
_K = _MIN_KEPT + 1
_C = 19
_B, _H, _W = 8, 512, 512
_CH = 128
_STEPS = _B * (_H // _CH)
_N = _B * _H * _W

_NW = 32
_LANES = 16
_WSL = _N // _NW
_NB1 = 2048
_NB3 = 1024


def _nll_kernel(pred_ref, tgt_ref, tfloor_ref, keys_ref, stats_ref):
    b = pl.program_id(0)
    h = pl.program_id(1)

    tgt = tgt_ref[0]
    x0 = pred_ref[0, 0]
    m = x0
    for c in range(1, _C):
        m = jnp.maximum(m, pred_ref[0, c])
    z = jnp.exp(x0 - m)
    tl = jnp.where(tgt == 0, x0, 0.0)
    for c in range(1, _C):
        xc = pred_ref[0, c]
        z = z + jnp.exp(xc - m)
        tl = tl + jnp.where(tgt == c, xc, 0.0)
    nll = m + jnp.log(z) - tl
    valid = tgt != _IGNORE
    keys_ref[0] = jnp.where(valid, nll, -1.0)

    tfloor = tfloor_ref[0]
    nv = jnp.sum(valid.astype(jnp.float32))
    asum = jnp.sum(jnp.where(valid, nll, 0.0))
    tfmask = jnp.logical_and(valid, nll >= tfloor)
    tfcnt = jnp.sum(tfmask.astype(jnp.float32))
    tfsum = jnp.sum(jnp.where(tfmask, nll, 0.0))

    @pl.when(jnp.logical_and(b == 0, h == 0))
    def _init():
        stats_ref[0] = 0.0
        stats_ref[1] = 0.0
        stats_ref[2] = 0.0
        stats_ref[3] = 0.0

    stats_ref[0] += nv
    stats_ref[1] += asum
    stats_ref[2] += tfcnt
    stats_ref[3] += tfsum


def _run_nll(pred, target, tfloor):
    return pl.pallas_call(
        _nll_kernel,
        grid=(_B, _H // _CH),
        in_specs=[
            pl.BlockSpec((1, _C, _CH, _W), lambda b, h: (b, 0, h, 0)),
            pl.BlockSpec((1, _CH, _W), lambda b, h: (b, h, 0)),
            pl.BlockSpec(memory_space=pltpu.SMEM),
        ],
        out_specs=[
            pl.BlockSpec((1, _CH, _W), lambda b, h: (b * (_H // _CH) + h, 0, 0)),
            pl.BlockSpec(memory_space=pltpu.SMEM),
        ],
        out_shape=[
            jax.ShapeDtypeStruct((_STEPS, _CH, _W), jnp.float32),
            jax.ShapeDtypeStruct((4,), jnp.float32),
        ],
    )(pred, target, tfloor)


_sc_mesh = plsc.VectorSubcoreMesh(core_axis_name="c", subcore_axis_name="s")
_sc_params = pltpu.CompilerParams(needs_layout_passes=False)


def _worker_id():
    return lax.axis_index("s") * 2 + lax.axis_index("c")


def _memset_zero(ref, nwords):
    zero = jnp.zeros((_LANES,), jnp.int32)

    def body(i, c):
        for j in range(4):
            ref[pl.ds((i * 4 + j) * _LANES, _LANES)] = zero
        return c

    lax.fori_loop(0, nwords // (4 * _LANES), body, jnp.int32(0))


def _lane_reduce(hist_v, red_v, nb):
    def body(k, c):
        acc = hist_v[pl.ds(k * _LANES, _LANES)]
        for l in range(1, _LANES):
            acc = acc + hist_v[pl.ds(l * nb + k * _LANES, _LANES)]
        red_v[pl.ds(k * _LANES, _LANES)] = acc
        return c

    lax.fori_loop(0, nb // _LANES, body, jnp.int32(0))


@functools.partial(
    pl.kernel, mesh=_sc_mesh, compiler_params=_sc_params,
    out_type=jax.ShapeDtypeStruct((_NW, _NB1), jnp.int32),
    scratch_types=[
        pltpu.VMEM((_NB1 * _LANES,), jnp.int32),
        pltpu.VMEM((_WSL,), jnp.float32),
        pltpu.VMEM((_NB1,), jnp.int32),
    ],
)
def _sc_scan1(keys_hbm, hist_out, hist_v, keys_v, red_v):
    wid = _worker_id()
    pltpu.sync_copy(keys_hbm.at[pl.ds(wid * _WSL, _WSL)], keys_v)
    _memset_zero(hist_v, _NB1 * _LANES)
    lane_off = lax.iota(jnp.int32, _LANES) * _NB1
    ones = jnp.ones((_LANES,), jnp.int32)

    def body(i, c):
        for j in range(4):
            v = keys_v[pl.ds(i * 64 + j * _LANES, _LANES)]
            ib = lax.bitcast_convert_type(v, jnp.int32)
            b1 = lax.shift_right_arithmetic(ib, 21)
            plsc.addupdate_scatter(hist_v, [lane_off + b1], ones,
                                   mask=b1 >= 0)
        return c

    lax.fori_loop(0, _WSL // 64, body, jnp.int32(0))
    _lane_reduce(hist_v, red_v, _NB1)
    pltpu.sync_copy(red_v, hist_out.at[wid])


@functools.partial(
    pl.kernel, mesh=_sc_mesh, compiler_params=_sc_params,
    out_type=jax.ShapeDtypeStruct((_NW, _NB1), jnp.int32),
    scratch_types=[
        pltpu.VMEM((_NB1 * _LANES,), jnp.int32),
        pltpu.VMEM((_WSL,), jnp.float32),
        pltpu.VMEM((_NB1,), jnp.int32),
        pltpu.VMEM((_LANES,), jnp.int32),
    ],
)
def _sc_scan2(keys_hbm, params_hbm, hist_out, hist_v, keys_v, red_v, par_v):
    wid = _worker_id()
    pltpu.sync_copy(keys_hbm.at[pl.ds(wid * _WSL, _WSL)], keys_v)
    pltpu.sync_copy(params_hbm, par_v)
    _memset_zero(hist_v, _NB1 * _LANES)
    b1 = par_v[...][0]
    lane_off = lax.iota(jnp.int32, _LANES) * _NB1
    ones = jnp.ones((_LANES,), jnp.int32)

    def body(i, c):
        for j in range(4):
            v = keys_v[pl.ds(i * 64 + j * _LANES, _LANES)]
            ib = lax.bitcast_convert_type(v, jnp.int32)
            sel = lax.shift_right_arithmetic(ib, 21) == b1
            bin2 = jnp.bitwise_and(lax.shift_right_logical(ib, 10), 0x7FF)
            plsc.addupdate_scatter(hist_v, [lane_off + bin2], ones, mask=sel)
        return c

    lax.fori_loop(0, _WSL // 64, body, jnp.int32(0))
    _lane_reduce(hist_v, red_v, _NB1)
    pltpu.sync_copy(red_v, hist_out.at[wid])


@functools.partial(
    pl.kernel, mesh=_sc_mesh, compiler_params=_sc_params,
    out_type=[
        jax.ShapeDtypeStruct((_NW, _NB3), jnp.int32),
        jax.ShapeDtypeStruct((_NW, _LANES), jnp.float32),
        jax.ShapeDtypeStruct((_NW, _LANES), jnp.int32),
    ],
    scratch_types=[
        pltpu.VMEM((_NB3 * _LANES,), jnp.int32),
        pltpu.VMEM((_WSL,), jnp.float32),
        pltpu.VMEM((_NB3,), jnp.int32),
        pltpu.VMEM((_LANES,), jnp.int32),
        pltpu.VMEM((_LANES,), jnp.float32),
        pltpu.VMEM((_LANES,), jnp.int32),
    ],
)
def _sc_scan3(keys_hbm, params_hbm, hist_out, gsum_out, gcnt_out,
              hist_v, keys_v, red_v, par_v, gsum_v, gcnt_v):
    wid = _worker_id()
    pltpu.sync_copy(keys_hbm.at[pl.ds(wid * _WSL, _WSL)], keys_v)
    pltpu.sync_copy(params_hbm, par_v)
    _memset_zero(hist_v, _NB3 * _LANES)
    p22 = par_v[...][0]
    lane_off = lax.iota(jnp.int32, _LANES) * _NB3
    ones = jnp.ones((_LANES,), jnp.int32)
    fzero = jnp.zeros((_LANES,), jnp.float32)
    izero = jnp.zeros((_LANES,), jnp.int32)

    def body(i, carry):
        gsum, gcnt = carry
        for j in range(4):
            v = keys_v[pl.ds(i * 64 + j * _LANES, _LANES)]
            ib = lax.bitcast_convert_type(v, jnp.int32)
            top22 = lax.shift_right_arithmetic(ib, 10)
            sel = top22 == p22
            bin3 = jnp.bitwise_and(ib, 0x3FF)
            plsc.addupdate_scatter(hist_v, [lane_off + bin3], ones, mask=sel)
            above = top22 > p22
            gsum = gsum + jnp.where(above, v, fzero)
            gcnt = gcnt + jnp.where(above, ones, izero)
        return gsum, gcnt

    gsum, gcnt = lax.fori_loop(0, _WSL // 64, body, (fzero, izero))
    gsum_v[...] = gsum
    gcnt_v[...] = gcnt
    _lane_reduce(hist_v, red_v, _NB3)
    pltpu.sync_copy(red_v, hist_out.at[wid])
    pltpu.sync_copy(gsum_v, gsum_out.at[wid])
    pltpu.sync_copy(gcnt_v, gcnt_out.at[wid])


def _count_ge(hist, binidx, cand):
    return jnp.sum(jnp.where(binidx >= cand, hist, 0))


def _locate_bits(hist, binidx, nbits, k):
    def body(i, v):
        bit = lax.shift_left(jnp.int32(1), nbits - 1 - i)
        cand = v | bit
        cnt = _count_ge(hist, binidx, cand)
        return jnp.where(cnt >= k, cand, v)

    return lax.fori_loop(0, nbits, body, jnp.int32(0))


def _locate1_kernel(hist_ref, out_ref):
    hist = hist_ref[...]
    binidx = lax.broadcasted_iota(jnp.int32, hist.shape, 1)
    k = jnp.int32(_K)
    b1 = _locate_bits(hist, binidx, 11, k)
    cnt_above = _count_ge(hist, binidx, b1 + 1)
    out_ref[0] = b1
    out_ref[1] = k - cnt_above


def _locate2_kernel(hist_ref, par_ref, out_ref):
    hist = hist_ref[...]
    binidx = lax.broadcasted_iota(jnp.int32, hist.shape, 1)
    b1 = par_ref[0]
    k2 = par_ref[1]
    b2 = _locate_bits(hist, binidx, 11, k2)
    cnt_above = _count_ge(hist, binidx, b2 + 1)
    out_ref[0] = lax.shift_left(b1, 11) | b2
    out_ref[1] = k2 - cnt_above


def _final_kernel(hist_ref, gsum_ref, gcnt_ref, par_ref, stats_ref, out_ref):
    hist = hist_ref[...]
    binidx = lax.broadcasted_iota(jnp.int32, hist.shape, 1)
    p22 = par_ref[0]
    k3 = par_ref[1]
    b3 = _locate_bits(hist, binidx, 10, k3)

    histf = hist.astype(jnp.float32)
    inbin_mask = jnp.logical_and(binidx >= b3, hist > 0)
    inbin_cnt = jnp.sum(jnp.where(inbin_mask, histf, 0.0))
    vals = lax.bitcast_convert_type(
        lax.shift_left(p22, 10) | binidx, jnp.float32)
    inbin_sum = jnp.sum(jnp.where(inbin_mask, vals * histf, 0.0))

    gt_cnt = jnp.sum(gcnt_ref[...]).astype(jnp.float32)
    gt_sum = jnp.sum(gsum_ref[...])

    kept_cnt = jnp.maximum(gt_cnt + inbin_cnt, 1.0)
    kept_sum = gt_sum + inbin_sum
    ohem_mean = kept_sum / kept_cnt

    num_valid = stats_ref[0]
    all_sum = stats_ref[1]
    tf_cnt = stats_ref[2]
    tf_sum = stats_ref[3]
    floor_mean = tf_sum / jnp.maximum(tf_cnt, 1.0)
    all_mean = all_sum / jnp.maximum(num_valid, 1.0)

    kept_mean = jnp.where(tf_cnt >= jnp.float32(_K), ohem_mean, floor_mean)
    out_ref[0] = jnp.where(num_valid > jnp.float32(_MIN_KEPT),
                           kept_mean, all_mean)


def _run_locate1(hist):
    return pl.pallas_call(
        _locate1_kernel,
        in_specs=[pl.BlockSpec((_NW, _NB1), lambda: (0, 0))],
        out_specs=pl.BlockSpec(memory_space=pltpu.SMEM),
        out_shape=jax.ShapeDtypeStruct((16,), jnp.int32),
    )(hist)


def _run_locate2(hist, params):
    return pl.pallas_call(
        _locate2_kernel,
        in_specs=[
            pl.BlockSpec((_NW, _NB1), lambda: (0, 0)),
            pl.BlockSpec(memory_space=pltpu.SMEM),
        ],
        out_specs=pl.BlockSpec(memory_space=pltpu.SMEM),
        out_shape=jax.ShapeDtypeStruct((16,), jnp.int32),
    )(hist, params)


def _run_final(hist3, gsum, gcnt, params, stats):
    return pl.pallas_call(
        _final_kernel,
        in_specs=[
            pl.BlockSpec((_NW, _NB3), lambda: (0, 0)),
            pl.BlockSpec((_NW, _LANES), lambda: (0, 0)),
            pl.BlockSpec((_NW, _LANES), lambda: (0, 0)),
            pl.BlockSpec(memory_space=pltpu.SMEM),
            pl.BlockSpec(memory_space=pltpu.SMEM),
        ],
        out_specs=pl.BlockSpec(memory_space=pltpu.SMEM),
        out_shape=jax.ShapeDtypeStruct((1,), jnp.float32),
    )(hist3, gsum, gcnt, params, stats)


@jax.jit
def kernel(pred, target):
    tfloor = -jnp.log(jnp.float32(_THRESH)).reshape(1)
    keys3d, stats = _run_nll(pred, target, tfloor)
    keys = keys3d.reshape(_N)

    hist1 = _sc_scan1(keys)
    par1 = _run_locate1(hist1)
    hist2 = _sc_scan2(keys, par1)
    par2 = _run_locate2(hist2, par1)
    hist3, gsum, gcnt = _sc_scan3(keys, par2)
    out = _run_final(hist3, gsum, gcnt, par2, stats)
    return out[0]


# speedup vs baseline: 1.5335x; 1.0507x over previous
import functools

import jax
import jax.numpy as jnp
from jax import lax
from jax.experimental import pallas as pl
from jax.experimental.pallas import tpu as pltpu
from jax.experimental.pallas import tpu_sc as plsc

_IGNORE = 255
_THRESH = 0.7
_MIN_KEPT = 100000
_K = _MIN_KEPT + 1
_C = 19
_B, _H, _W = 8, 512, 512
_CH = 128
_STEPS = _B * (_H // _CH)
_N = _B * _H * _W

_NW = 32
_LANES = 16
_WSL = _N // _NW
_NB1 = 2048
_NB3 = 1024


def _nll_kernel(pred_ref, tgt_ref, tfloor_ref, keys_ref, stats_ref):
    b = pl.program_id(0)
    h = pl.program_id(1)

    tgt = tgt_ref[0]
    x0 = pred_ref[0, 0]
    m = x0
    for c in range(1, _C):
        m = jnp.maximum(m, pred_ref[0, c])
    z = jnp.exp(x0 - m)
    tl = jnp.where(tgt == 0, x0, 0.0)
    for c in range(1, _C):
        xc = pred_ref[0, c]
        z = z + jnp.exp(xc - m)
        tl = tl + jnp.where(tgt == c, xc, 0.0)
    nll = m + jnp.log(z) - tl
    valid = tgt != _IGNORE
    keys_ref[0] = jnp.where(valid, nll, -1.0)

    tfloor = tfloor_ref[0]
    nv = jnp.sum(valid.astype(jnp.float32))
    asum = jnp.sum(jnp.where(valid, nll, 0.0))
    tfmask = jnp.logical_and(valid, nll >= tfloor)
    tfcnt = jnp.sum(tfmask.astype(jnp.float32))
    tfsum = jnp.sum(jnp.where(tfmask, nll, 0.0))

    @pl.when(jnp.logical_and(b == 0, h == 0))
    def _init():
        stats_ref[0] = 0.0
        stats_ref[1] = 0.0
        stats_ref[2] = 0.0
        stats_ref[3] = 0.0

    stats_ref[0] += nv
    stats_ref[1] += asum
    stats_ref[2] += tfcnt
    stats_ref[3] += tfsum


def _run_nll(pred, target, tfloor):
    return pl.pallas_call(
        _nll_kernel,
        grid=(_B, _H // _CH),
        in_specs=[
            pl.BlockSpec((1, _C, _CH, _W), lambda b, h: (b, 0, h, 0)),
            pl.BlockSpec((1, _CH, _W), lambda b, h: (b, h, 0)),
            pl.BlockSpec(memory_space=pltpu.SMEM),
        ],
        out_specs=[
            pl.BlockSpec((1, _CH, _W), lambda b, h: (b * (_H // _CH) + h, 0, 0)),
            pl.BlockSpec(memory_space=pltpu.SMEM),
        ],
        out_shape=[
            jax.ShapeDtypeStruct((_STEPS, _CH, _W), jnp.float32),
            jax.ShapeDtypeStruct((4,), jnp.float32),
        ],
    )(pred, target, tfloor)


_sc_mesh = plsc.VectorSubcoreMesh(core_axis_name="c", subcore_axis_name="s")
_sc_params = pltpu.CompilerParams(needs_layout_passes=False)


def _worker_id():
    return lax.axis_index("s") * 2 + lax.axis_index("c")


def _scan_rows(keys_hbm, keys_v, wid, vec_body, carry0):
    pltpu.sync_copy(keys_hbm.at[wid], keys_v)

    def row(r, carry):
        for j in range(_W // _LANES):
            v = keys_v[r, pl.ds(j * _LANES, _LANES)]
            ib = lax.bitcast_convert_type(v, jnp.int32)
            carry = vec_body(v, ib, carry)
        return carry

    return lax.fori_loop(0, _CH, row, carry0)


def _memset_zero(ref, nwords):
    zero = jnp.zeros((_LANES,), jnp.int32)

    def body(i, c):
        for j in range(4):
            ref[pl.ds((i * 4 + j) * _LANES, _LANES)] = zero
        return c

    lax.fori_loop(0, nwords // (4 * _LANES), body, jnp.int32(0))


def _lane_reduce(hist_v, red_v, nb):
    def body(k, c):
        acc = hist_v[pl.ds(k * _LANES, _LANES)]
        for l in range(1, _LANES):
            acc = acc + hist_v[pl.ds(l * nb + k * _LANES, _LANES)]
        red_v[pl.ds(k * _LANES, _LANES)] = acc
        return c

    lax.fori_loop(0, nb // _LANES, body, jnp.int32(0))


@functools.partial(
    pl.kernel, mesh=_sc_mesh, compiler_params=_sc_params,
    out_type=jax.ShapeDtypeStruct((_NW, _NB1), jnp.int32),
    scratch_types=[
        pltpu.VMEM((_NB1 * _LANES,), jnp.int32),
        pltpu.VMEM((_CH, _W), jnp.float32),
        pltpu.VMEM((_NB1,), jnp.int32),
    ],
)
def _sc_scan1(keys_hbm, hist_out, hist_v, keys_v, red_v):
    wid = _worker_id()
    _memset_zero(hist_v, _NB1 * _LANES)
    lane_off = lax.iota(jnp.int32, _LANES) * _NB1
    ones = jnp.ones((_LANES,), jnp.int32)

    def body(v, ib, c):
        b1 = lax.shift_right_arithmetic(ib, 21)
        plsc.addupdate_scatter(hist_v, [lane_off + b1], ones, mask=b1 >= 0)
        return c

    _scan_rows(keys_hbm, keys_v, wid, body, jnp.int32(0))
    _lane_reduce(hist_v, red_v, _NB1)
    pltpu.sync_copy(red_v, hist_out.at[wid])


@functools.partial(
    pl.kernel, mesh=_sc_mesh, compiler_params=_sc_params,
    out_type=jax.ShapeDtypeStruct((_NW, _NB1), jnp.int32),
    scratch_types=[
        pltpu.VMEM((_NB1 * _LANES,), jnp.int32),
        pltpu.VMEM((_CH, _W), jnp.float32),
        pltpu.VMEM((_NB1,), jnp.int32),
        pltpu.VMEM((_LANES,), jnp.int32),
    ],
)
def _sc_scan2(keys_hbm, params_hbm, hist_out, hist_v, keys_v, red_v, par_v):
    wid = _worker_id()
    pltpu.sync_copy(params_hbm, par_v)
    _memset_zero(hist_v, _NB1 * _LANES)
    b1 = par_v[...][0]
    lane_off = lax.iota(jnp.int32, _LANES) * _NB1
    ones = jnp.ones((_LANES,), jnp.int32)

    def body(v, ib, c):
        sel = lax.shift_right_arithmetic(ib, 21) == b1
        bin2 = jnp.bitwise_and(lax.shift_right_logical(ib, 10), 0x7FF)
        plsc.addupdate_scatter(hist_v, [lane_off + bin2], ones, mask=sel)
        return c

    _scan_rows(keys_hbm, keys_v, wid, body, jnp.int32(0))
    _lane_reduce(hist_v, red_v, _NB1)
    pltpu.sync_copy(red_v, hist_out.at[wid])


@functools.partial(
    pl.kernel, mesh=_sc_mesh, compiler_params=_sc_params,
    out_type=[
        jax.ShapeDtypeStruct((_NW, _NB3), jnp.int32),
        jax.ShapeDtypeStruct((_NW, _LANES), jnp.float32),
        jax.ShapeDtypeStruct((_NW, _LANES), jnp.int32),
    ],
    scratch_types=[
        pltpu.VMEM((_NB3 * _LANES,), jnp.int32),
        pltpu.VMEM((_CH, _W), jnp.float32),
        pltpu.VMEM((_NB3,), jnp.int32),
        pltpu.VMEM((_LANES,), jnp.int32),
        pltpu.VMEM((_LANES,), jnp.float32),
        pltpu.VMEM((_LANES,), jnp.int32),
    ],
)
def _sc_scan3(keys_hbm, params_hbm, hist_out, gsum_out, gcnt_out,
              hist_v, keys_v, red_v, par_v, gsum_v, gcnt_v):
    wid = _worker_id()
    pltpu.sync_copy(params_hbm, par_v)
    _memset_zero(hist_v, _NB3 * _LANES)
    p22 = par_v[...][0]
    lane_off = lax.iota(jnp.int32, _LANES) * _NB3
    ones = jnp.ones((_LANES,), jnp.int32)
    fzero = jnp.zeros((_LANES,), jnp.float32)
    izero = jnp.zeros((_LANES,), jnp.int32)

    def body(v, ib, carry):
        gsum, gcnt = carry
        top22 = lax.shift_right_arithmetic(ib, 10)
        sel = top22 == p22
        bin3 = jnp.bitwise_and(ib, 0x3FF)
        plsc.addupdate_scatter(hist_v, [lane_off + bin3], ones, mask=sel)
        above = top22 > p22
        gsum = gsum + jnp.where(above, v, fzero)
        gcnt = gcnt + jnp.where(above, ones, izero)
        return gsum, gcnt

    gsum, gcnt = _scan_rows(keys_hbm, keys_v, wid, body, (fzero, izero))
    gsum_v[...] = gsum
    gcnt_v[...] = gcnt
    _lane_reduce(hist_v, red_v, _NB3)
    pltpu.sync_copy(red_v, hist_out.at[wid])
    pltpu.sync_copy(gsum_v, gsum_out.at[wid])
    pltpu.sync_copy(gcnt_v, gcnt_out.at[wid])


def _count_ge(hist, binidx, cand):
    return jnp.sum(jnp.where(binidx >= cand, hist, 0))


def _locate_bits(hist, binidx, nbits, k):
    def body(i, v):
        bit = lax.shift_left(jnp.int32(1), nbits - 1 - i)
        cand = v | bit
        cnt = _count_ge(hist, binidx, cand)
        return jnp.where(cnt >= k, cand, v)

    return lax.fori_loop(0, nbits, body, jnp.int32(0))


def _locate1_kernel(hist_ref, out_ref):
    hist = hist_ref[...]
    binidx = lax.broadcasted_iota(jnp.int32, hist.shape, 1)
    k = jnp.int32(_K)
    b1 = _locate_bits(hist, binidx, 11, k)
    cnt_above = _count_ge(hist, binidx, b1 + 1)
    out_ref[0] = b1
    out_ref[1] = k - cnt_above


def _locate2_kernel(hist_ref, par_ref, out_ref):
    hist = hist_ref[...]
    binidx = lax.broadcasted_iota(jnp.int32, hist.shape, 1)
    b1 = par_ref[0]
    k2 = par_ref[1]
    b2 = _locate_bits(hist, binidx, 11, k2)
    cnt_above = _count_ge(hist, binidx, b2 + 1)
    out_ref[0] = lax.shift_left(b1, 11) | b2
    out_ref[1] = k2 - cnt_above


def _final_kernel(hist_ref, gsum_ref, gcnt_ref, par_ref, stats_ref, out_ref):
    hist = hist_ref[...]
    binidx = lax.broadcasted_iota(jnp.int32, hist.shape, 1)
    p22 = par_ref[0]
    k3 = par_ref[1]
    b3 = _locate_bits(hist, binidx, 10, k3)

    histf = hist.astype(jnp.float32)
    inbin_mask = jnp.logical_and(binidx >= b3, hist > 0)
    inbin_cnt = jnp.sum(jnp.where(inbin_mask, histf, 0.0))
    vals = lax.bitcast_convert_type(
        lax.shift_left(p22, 10) | binidx, jnp.float32)
    inbin_sum = jnp.sum(jnp.where(inbin_mask, vals * histf, 0.0))

    gt_cnt = jnp.sum(gcnt_ref[...]).astype(jnp.float32)
    gt_sum = jnp.sum(gsum_ref[...])

    kept_cnt = jnp.maximum(gt_cnt + inbin_cnt, 1.0)
    kept_sum = gt_sum + inbin_sum
    ohem_mean = kept_sum / kept_cnt

    num_valid = stats_ref[0]
    all_sum = stats_ref[1]
    tf_cnt = stats_ref[2]
    tf_sum = stats_ref[3]
    floor_mean = tf_sum / jnp.maximum(tf_cnt, 1.0)
    all_mean = all_sum / jnp.maximum(num_valid, 1.0)

    kept_mean = jnp.where(tf_cnt >= jnp.float32(_K), ohem_mean, floor_mean)
    out_ref[0] = jnp.where(num_valid > jnp.float32(_MIN_KEPT),
                           kept_mean, all_mean)


def _run_locate1(hist):
    return pl.pallas_call(
        _locate1_kernel,
        in_specs=[pl.BlockSpec((_NW, _NB1), lambda: (0, 0))],
        out_specs=pl.BlockSpec(memory_space=pltpu.SMEM),
        out_shape=jax.ShapeDtypeStruct((16,), jnp.int32),
    )(hist)


def _run_locate2(hist, params):
    return pl.pallas_call(
        _locate2_kernel,
        in_specs=[
            pl.BlockSpec((_NW, _NB1), lambda: (0, 0)),
            pl.BlockSpec(memory_space=pltpu.SMEM),
        ],
        out_specs=pl.BlockSpec(memory_space=pltpu.SMEM),
        out_shape=jax.ShapeDtypeStruct((16,), jnp.int32),
    )(hist, params)


def _run_final(hist3, gsum, gcnt, params, stats):
    return pl.pallas_call(
        _final_kernel,
        in_specs=[
            pl.BlockSpec((_NW, _NB3), lambda: (0, 0)),
            pl.BlockSpec((_NW, _LANES), lambda: (0, 0)),
            pl.BlockSpec((_NW, _LANES), lambda: (0, 0)),
            pl.BlockSpec(memory_space=pltpu.SMEM),
            pl.BlockSpec(memory_space=pltpu.SMEM),
        ],
        out_specs=pl.BlockSpec(memory_space=pltpu.SMEM),
        out_shape=jax.ShapeDtypeStruct((1,), jnp.float32),
    )(hist3, gsum, gcnt, params, stats)


@jax.jit
def kernel(pred, target):
    tfloor = -jnp.log(jnp.float32(_THRESH)).reshape(1)
    keys, stats = _run_nll(pred, target, tfloor)

    hist1 = _sc_scan1(keys)
    par1 = _run_locate1(hist1)
    hist2 = _sc_scan2(keys, par1)
    par2 = _run_locate2(hist2, par1)
    hist3, gsum, gcnt = _sc_scan3(keys, par2)
    out = _run_final(hist3, gsum, gcnt, par2, stats)
    return out[0]
